# Initial kernel scaffold; baseline (speedup 1.0000x reference)
import functools

import jax
import jax.numpy as jnp
from jax import lax
from jax.experimental import pallas as pl
from jax.experimental.pallas import tpu as pltpu
from jax.experimental.pallas import tpu_sc as plsc

N = 10000
E = 320000
D = 128
DE = 144
NC = 2
NS = 16
NW = NC * NS
PER_W = E // NW
K = 80
NCHUNK = PER_W // K
ROWS_PER_TILE = N // NS
ZB = 125


def _linear_kernel(x_ref, wt_ref, b_ref, out_ref):
    y = jnp.dot(x_ref[...], wt_ref[...], preferred_element_type=jnp.float32)
    out_ref[:, :D] = y + b_ref[...]
    col16 = lax.broadcasted_iota(jnp.int32, (x_ref.shape[0], 16), 1)
    out_ref[:, D:] = jnp.where(col16 == 0, 1.0, 0.0)


def _combine_kernel(yext_ref, agg_ref, ms_ref, out_ref):
    stot = agg_ref[0] + agg_ref[1]
    deg = jnp.maximum(stot[:, D:D + 1], 1.0)
    out_ref[...] = yext_ref[:, :D] + ms_ref[0, 0] * (stot[:, :D] / deg)


def _sc_agg_body(yext_hbm, col_hbm, row_hbm, w_hbm, out_hbm,
                 col_v, row_v, w_v, rows_v, zbuf, accum, sem):
    c = lax.axis_index("c")
    s = lax.axis_index("s")
    wid = s * NC + c

    zeros16 = jnp.zeros((16,), jnp.float32)

    def zrow(i, _):
        for d in range(DE // 16):
            zbuf[i, pl.ds(d * 16, 16)] = zeros16
        return 0

    lax.fori_loop(0, ZB, zrow, 0)
    tile_row0 = s * ROWS_PER_TILE
    for j in range(ROWS_PER_TILE // ZB):
        pltpu.sync_copy(zbuf, accum.at[pl.ds(tile_row0 + j * ZB, ZB)])
    plsc.subcore_barrier()

    base_e0 = wid * PER_W

    def chunk(ci, _):
        base = pl.multiple_of(base_e0 + ci * K, 8)
        pltpu.sync_copy(col_hbm.at[pl.ds(base, K)], col_v)
        pltpu.sync_copy(row_hbm.at[pl.ds(base, K)], row_v)
        pltpu.sync_copy(w_hbm.at[pl.ds(base, K)], w_v)
        pltpu.async_copy(yext_hbm.at[col_v], rows_v, sem).wait()

        def scale(e, _):
            we = w_v[e]
            for d in range(D // 16):
                sl = pl.ds(d * 16, 16)
                rows_v[e, sl] = rows_v[e, sl] * we
            return 0

        lax.fori_loop(0, K, scale, 0)
        pltpu.sync_copy(rows_v, accum.at[row_v], add=True)
        return 0

    lax.fori_loop(0, NCHUNK, chunk, 0)
    plsc.subcore_barrier()

    for j in range(ROWS_PER_TILE // ZB):
        r0 = tile_row0 + j * ZB
        pltpu.sync_copy(accum.at[pl.ds(r0, ZB)], out_hbm.at[c, pl.ds(r0, ZB)])


_sc_agg = functools.partial(
    pl.kernel,
    out_type=jax.ShapeDtypeStruct((NC, N, DE), jnp.float32),
    mesh=plsc.VectorSubcoreMesh(core_axis_name="c", subcore_axis_name="s",
                                num_cores=NC, num_subcores=NS),
    scratch_types=[
        pltpu.VMEM((K,), jnp.int32),
        pltpu.VMEM((K,), jnp.int32),
        pltpu.VMEM((K,), jnp.float32),
        pltpu.VMEM((K, DE), jnp.float32),
        pltpu.VMEM((ZB, DE), jnp.float32),
        pltpu.VMEM_SHARED((N, DE), jnp.float32),
        pltpu.SemaphoreType.DMA,
    ],
)(_sc_agg_body)


def kernel(x, edge_index, edge_metric, W, b, metric_scale):
    row = edge_index[0].astype(jnp.int32)
    col = edge_index[1].astype(jnp.int32)
    wt = W.T
    b2 = b.reshape(1, D)
    ms2 = metric_scale.reshape(1, 1)

    blk = 1000
    grid = (N // blk,)
    yext = pl.pallas_call(
        _linear_kernel,
        grid=grid,
        in_specs=[
            pl.BlockSpec((blk, D), lambda i: (i, 0)),
            pl.BlockSpec((D, D), lambda i: (0, 0)),
            pl.BlockSpec((1, D), lambda i: (0, 0)),
        ],
        out_specs=pl.BlockSpec((blk, DE), lambda i: (i, 0)),
        out_shape=jax.ShapeDtypeStruct((N, DE), jnp.float32),
    )(x, wt, b2)

    agg = _sc_agg(yext, col, row, edge_metric)

    out = pl.pallas_call(
        _combine_kernel,
        grid=grid,
        in_specs=[
            pl.BlockSpec((blk, DE), lambda i: (i, 0)),
            pl.BlockSpec((NC, blk, DE), lambda i: (0, i, 0)),
            pl.BlockSpec(memory_space=pltpu.SMEM),
        ],
        out_specs=pl.BlockSpec((blk, D), lambda i: (i, 0)),
        out_shape=jax.ShapeDtypeStruct((N, D), jnp.float32),
    )(x, agg, ms2)
    return out


# speedup vs baseline: 4.0526x; 4.0526x over previous
import functools

import jax
import jax.numpy as jnp
from jax import lax
from jax.experimental import pallas as pl
from jax.experimental.pallas import tpu as pltpu
from jax.experimental.pallas import tpu_sc as plsc

N = 10000
E = 320000
D = 128
DE = 144
NC = 2
NS = 16
NW = NC * NS
PER_W = E // NW
K = 80
NCHUNK = PER_W // K
NP_ = 10240
ROWS_PER_TILE = NP_ // NS
ZB = 128


def _linear_kernel(x_ref, wt_ref, b_ref, out_ref):
    y = jnp.dot(x_ref[...], wt_ref[...], preferred_element_type=jnp.float32)
    out_ref[:, :D] = y + b_ref[...]
    col16 = lax.broadcasted_iota(jnp.int32, (x_ref.shape[0], 16), 1)
    out_ref[:, D:] = jnp.where(col16 == 0, 1.0, 0.0)


def _combine_kernel(yext_ref, agg_ref, ms_ref, out_ref):
    stot = agg_ref[0] + agg_ref[1]
    deg = jnp.maximum(stot[:, D:D + 1], 1.0)
    out_ref[...] = yext_ref[:, :D] + ms_ref[0, 0] * (stot[:, :D] / deg)


def _sc_agg_body(yext_hbm, col_hbm, row_hbm, w_hbm, out_hbm,
                 col_v, row_v, w_v, rows_v, zbuf, accum, sem):
    c = lax.axis_index("c")
    s = lax.axis_index("s")
    wid = s * NC + c

    zeros16 = jnp.zeros((16,), jnp.float32)

    def zrow(i, _):
        for d in range(DE // 16):
            zbuf[i, pl.ds(d * 16, 16)] = zeros16
        return 0

    lax.fori_loop(0, ZB, zrow, 0)
    tile_row0 = s * ROWS_PER_TILE
    for j in range(ROWS_PER_TILE // ZB):
        pltpu.sync_copy(zbuf, accum.at[pl.ds(tile_row0 + j * ZB, ZB)])
    plsc.subcore_barrier()

    base_e0 = wid * PER_W

    def chunk(ci, _):
        base = pl.multiple_of(base_e0 + ci * K, 8)
        pltpu.sync_copy(col_hbm.at[pl.ds(base, K)], col_v)
        pltpu.sync_copy(row_hbm.at[pl.ds(base, K)], row_v)
        pltpu.sync_copy(w_hbm.at[pl.ds(base, K)], w_v)
        pltpu.async_copy(yext_hbm.at[col_v], rows_v, sem).wait()

        def scale(g, _):
            wv = w_v[pl.ds(g * 16, 16)]
            for i in range(16):
                we = wv[i]
                e = g * 16 + i
                for d in range(D // 16):
                    sl = pl.ds(d * 16, 16)
                    rows_v[e, sl] = rows_v[e, sl] * we
            return 0

        lax.fori_loop(0, K // 16, scale, 0)
        pltpu.sync_copy(rows_v, accum.at[row_v], add=True)
        return 0

    lax.fori_loop(0, NCHUNK, chunk, 0)
    plsc.subcore_barrier()

    for j in range(ROWS_PER_TILE // ZB):
        r0 = tile_row0 + j * ZB
        pltpu.sync_copy(accum.at[pl.ds(r0, ZB)], out_hbm.at[c, pl.ds(r0, ZB)])


_sc_agg = functools.partial(
    pl.kernel,
    out_type=jax.ShapeDtypeStruct((NC, NP_, DE), jnp.float32),
    mesh=plsc.VectorSubcoreMesh(core_axis_name="c", subcore_axis_name="s",
                                num_cores=NC, num_subcores=NS),
    compiler_params=pltpu.CompilerParams(use_tc_tiling_on_sc=False),
    scratch_types=[
        pltpu.VMEM((K,), jnp.int32),
        pltpu.VMEM((K,), jnp.int32),
        pltpu.VMEM((K,), jnp.float32),
        pltpu.VMEM((K, DE), jnp.float32),
        pltpu.VMEM((ZB, DE), jnp.float32),
        pltpu.VMEM_SHARED((NP_, DE), jnp.float32),
        pltpu.SemaphoreType.DMA,
    ],
)(_sc_agg_body)


def kernel(x, edge_index, edge_metric, W, b, metric_scale):
    row = edge_index[0].astype(jnp.int32)
    col = edge_index[1].astype(jnp.int32)
    wt = W.T
    b2 = b.reshape(1, D)
    ms2 = metric_scale.reshape(1, 1)

    blk = 1000
    grid = (N // blk,)
    yext = pl.pallas_call(
        _linear_kernel,
        grid=grid,
        in_specs=[
            pl.BlockSpec((blk, D), lambda i: (i, 0)),
            pl.BlockSpec((D, D), lambda i: (0, 0)),
            pl.BlockSpec((1, D), lambda i: (0, 0)),
        ],
        out_specs=pl.BlockSpec((blk, DE), lambda i: (i, 0)),
        out_shape=jax.ShapeDtypeStruct((N, DE), jnp.float32),
    )(x, wt, b2)

    agg = _sc_agg(yext, col, row, edge_metric)

    out = pl.pallas_call(
        _combine_kernel,
        grid=grid,
        in_specs=[
            pl.BlockSpec((blk, DE), lambda i: (i, 0)),
            pl.BlockSpec((NC, blk, DE), lambda i: (0, i, 0)),
            pl.BlockSpec(memory_space=pltpu.SMEM),
        ],
        out_specs=pl.BlockSpec((blk, D), lambda i: (i, 0)),
        out_shape=jax.ShapeDtypeStruct((N, D), jnp.float32),
    )(yext, agg, ms2)
    return out
